# baseline (speedup 1.0000x reference)
import functools

import jax
import jax.numpy as jnp
from jax import lax
from jax.experimental import pallas as pl
from jax.experimental.pallas import tpu as pltpu
from jax.experimental.pallas import tpu_sc as plsc

N, D_IN, D_OUT, E, ES = 4096, 1024, 1024, 16, 2
TOK_TILE = 256
TK = 128
G = (ES * N) // TK + E
PADDED = G * TK
NSLOT = ES * N
CS_B = 512
CS_NB = NSLOT // CS_B

NWORKERS = 32
CHUNK = 16


def _router_body(x_ref, ds_ref, c_ref, b_ref,
                 p1_ref, p2_ref, xw0_ref, xw1_ref, bias_ref):
    xb = x_ref[...]
    s = ds_ref[0, 0]
    cols = []
    for e in range(E):
        diff = s * (xb - c_ref[e, :][None, :])
        prod = diff * diff
        acc = jnp.sum(prod[:, 0:128], axis=1)
        for k in range(1, D_IN // 128):
            acc = acc + jnp.sum(prod[:, 128 * k:128 * (k + 1)], axis=1)
        cols.append(acc * (1.0 / D_IN))
    dist = jnp.stack(cols, axis=1)

    eids = lax.broadcasted_iota(jnp.int32, dist.shape, 1)
    m1 = jnp.min(dist, axis=1, keepdims=True)
    p1 = jnp.min(jnp.where(dist == m1, eids, E), axis=1)
    dist2 = jnp.where(eids == p1[:, None], jnp.inf, dist)
    m2 = jnp.min(dist2, axis=1, keepdims=True)
    p2 = jnp.min(jnp.where(dist2 == m2, eids, E), axis=1)

    sel = dist[:, 0:ES]
    wraw = 1.0 / (1.0 + sel)
    w = wraw / jnp.sum(wraw, axis=1, keepdims=True)

    p1_ref[...] = p1[:, None]
    p2_ref[...] = p2[:, None]
    xw0_ref[...] = w[:, 0:1] * xb
    xw1_ref[...] = w[:, 1:2] * xb
    g = (w[:, 0:1] * (p1[:, None] == eids).astype(jnp.float32)
         + w[:, 1:2] * (p2[:, None] == eids).astype(jnp.float32))
    bias_ref[...] = jnp.dot(g, b_ref[...], preferred_element_type=jnp.float32)


def _router(x, ds2, c2, b):
    n_tiles = N // TOK_TILE
    return pl.pallas_call(
        _router_body,
        grid=(n_tiles,),
        in_specs=[
            pl.BlockSpec((TOK_TILE, D_IN), lambda i: (i, 0)),
            pl.BlockSpec((1, 1), lambda i: (0, 0)),
            pl.BlockSpec((E, D_IN), lambda i: (0, 0)),
            pl.BlockSpec((E, D_OUT), lambda i: (0, 0)),
        ],
        out_specs=[
            pl.BlockSpec((TOK_TILE, 1), lambda i: (i, 0)),
            pl.BlockSpec((TOK_TILE, 1), lambda i: (i, 0)),
            pl.BlockSpec((TOK_TILE, D_IN), lambda i: (i, 0)),
            pl.BlockSpec((TOK_TILE, D_IN), lambda i: (i, 0)),
            pl.BlockSpec((TOK_TILE, D_OUT), lambda i: (i, 0)),
        ],
        out_shape=[
            jax.ShapeDtypeStruct((N, 1), jnp.int32),
            jax.ShapeDtypeStruct((N, 1), jnp.int32),
            jax.ShapeDtypeStruct((N, D_IN), jnp.float32),
            jax.ShapeDtypeStruct((N, D_IN), jnp.float32),
            jax.ShapeDtypeStruct((N, D_OUT), jnp.float32),
        ],
    )(x, ds2, c2, b)


def _book_body(eslot_ref, pos_ref, te_ref):
    eslot = eslot_ref[...]
    eids = lax.broadcasted_iota(jnp.int32, (NSLOT, E), 1)
    onehot = (eslot == eids).astype(jnp.float32)

    r512 = lax.broadcasted_iota(jnp.int32, (CS_B, CS_B), 0)
    c512 = lax.broadcasted_iota(jnp.int32, (CS_B, CS_B), 1)
    trilS = (c512 < r512).astype(jnp.float32)

    bsums = []
    for bidx in range(CS_NB):
        oh_b = onehot[bidx * CS_B:(bidx + 1) * CS_B, :]
        bsums.append(jnp.sum(oh_b, axis=0))
    block_sums = jnp.stack(bsums, axis=0)

    rB = lax.broadcasted_iota(jnp.int32, (CS_NB, CS_NB), 0)
    cB = lax.broadcasted_iota(jnp.int32, (CS_NB, CS_NB), 1)
    trilB = (cB < rB).astype(jnp.float32)
    block_excl = jnp.dot(trilB, block_sums,
                         preferred_element_type=jnp.float32)

    counts = jnp.sum(block_sums, axis=0)
    ntk = jnp.floor((counts + (TK - 1)) * (1.0 / TK)) * TK
    rE = lax.broadcasted_iota(jnp.int32, (E, E), 0)
    cE = lax.broadcasted_iota(jnp.int32, (E, E), 1)
    po = jnp.sum(jnp.where(rE < cE, ntk[:, None], 0.0), axis=0)
    end_off = po + ntk

    for bidx in range(CS_NB):
        oh_b = onehot[bidx * CS_B:(bidx + 1) * CS_B, :]
        within = jnp.dot(trilS, oh_b, preferred_element_type=jnp.float32)
        texcl = within + block_excl[bidx, :][None, :] + po[None, :]
        posb = jnp.sum(oh_b * texcl, axis=1)
        pos_ref[bidx * CS_B:(bidx + 1) * CS_B, :] = posb.astype(jnp.int32)[:, None]

    tvals = (lax.broadcasted_iota(jnp.int32, (1, G), 1) * TK).astype(jnp.float32)
    te = jnp.sum((tvals >= end_off[:, None]).astype(jnp.int32), axis=0)
    te_ref[...] = jnp.minimum(te, E - 1)[None, :]


def _bookkeeping(eslot_col):
    return pl.pallas_call(
        _book_body,
        in_specs=[pl.BlockSpec((NSLOT, 1), lambda: (0, 0))],
        out_specs=[
            pl.BlockSpec((NSLOT, 1), lambda: (0, 0)),
            pl.BlockSpec((1, G), lambda: (0, 0)),
        ],
        out_shape=[
            jax.ShapeDtypeStruct((NSLOT, 1), jnp.int32),
            jax.ShapeDtypeStruct((1, G), jnp.int32),
        ],
    )(eslot_col)


SCHUNK = 32


def _make_scatter():
    mesh = plsc.VectorSubcoreMesh(core_axis_name="c", subcore_axis_name="s")
    rows_per_w = N // (NWORKERS // ES)
    nchunk = rows_per_w // SCHUNK

    @functools.partial(
        pl.kernel, mesh=mesh,
        out_type=jax.ShapeDtypeStruct((PADDED, D_IN), jnp.float32),
        scratch_types=[
            pltpu.VMEM((SCHUNK, D_IN), jnp.float32),
            pltpu.VMEM((SCHUNK, D_IN), jnp.float32),
            pltpu.VMEM((SCHUNK,), jnp.int32),
            pltpu.VMEM((SCHUNK,), jnp.int32),
            pltpu.SemaphoreType.DMA,
            pltpu.SemaphoreType.DMA,
        ],
    )
    def scatter_k(xw0_hbm, xw1_hbm, pos_hbm, xs_hbm,
                  xbuf0, xbuf1, pbuf0, pbuf1, gsem, ssem):
        wid = lax.axis_index("c") * 16 + lax.axis_index("s")
        sid = wid // 16
        base = (wid % 16) * rows_per_w
        xbufs = (xbuf0, xbuf1)
        pbufs = (pbuf0, pbuf1)

        scat = [None, None]
        for ci in range(nchunk):
            b = ci % 2
            t0 = base + ci * SCHUNK
            if scat[b] is not None:
                scat[b].wait()

            @pl.when(sid == 0)
            def _a(t0=t0, b=b):
                pltpu.sync_copy(xw0_hbm.at[pl.ds(t0, SCHUNK), :], xbufs[b])
                pltpu.sync_copy(pos_hbm.at[0, pl.ds(t0, SCHUNK)], pbufs[b])

            @pl.when(sid == 1)
            def _b(t0=t0, b=b):
                pltpu.sync_copy(xw1_hbm.at[pl.ds(t0, SCHUNK), :], xbufs[b])
                pltpu.sync_copy(pos_hbm.at[1, pl.ds(t0, SCHUNK)], pbufs[b])

            scat[b] = pltpu.async_copy(xbufs[b], xs_hbm.at[pbufs[b]], ssem)
        scat[0].wait()
        scat[1].wait()

    return scatter_k


def _gemm_body(te_ref, xs_ref, w_ref, out_ref):
    out_ref[...] = lax.dot_general(xs_ref[...], w_ref[0],
                                   (((1,), (1,)), ((), ())))


def _grouped_gemm(xs, W, tile_expert):
    grid_spec = pltpu.PrefetchScalarGridSpec(
        num_scalar_prefetch=1,
        grid=(G,),
        in_specs=[
            pl.BlockSpec((TK, D_IN), lambda i, te: (i, 0)),
            pl.BlockSpec((1, D_OUT, D_IN), lambda i, te: (te[i], 0, 0)),
        ],
        out_specs=pl.BlockSpec((TK, D_OUT), lambda i, te: (i, 0)),
    )
    return pl.pallas_call(
        _gemm_body,
        grid_spec=grid_spec,
        out_shape=jax.ShapeDtypeStruct((PADDED, D_OUT), jnp.float32),
    )(tile_expert, xs, W)


CCHUNK = 16


def _make_combine():
    mesh = plsc.VectorSubcoreMesh(core_axis_name="c", subcore_axis_name="s")
    tok_per_w = N // NWORKERS
    nchunk = tok_per_w // CCHUNK

    @functools.partial(
        pl.kernel, mesh=mesh,
        out_type=jax.ShapeDtypeStruct((N, D_OUT), jnp.float32),
        scratch_types=[
            pltpu.VMEM((CCHUNK, D_OUT), jnp.float32),
            pltpu.VMEM((CCHUNK, D_OUT), jnp.float32),
            pltpu.VMEM((CCHUNK, D_OUT), jnp.float32),
            pltpu.VMEM((CCHUNK, D_OUT), jnp.float32),
            pltpu.VMEM((CCHUNK, D_OUT), jnp.float32),
            pltpu.VMEM((CCHUNK, D_OUT), jnp.float32),
            pltpu.VMEM((tok_per_w,), jnp.int32),
            pltpu.VMEM((tok_per_w,), jnp.int32),
            pltpu.SemaphoreType.DMA,
            pltpu.SemaphoreType.DMA,
            pltpu.SemaphoreType.DMA,
            pltpu.SemaphoreType.DMA,
        ],
    )
    def combine_k(y_hbm, pos_hbm, bias_hbm, out_hbm,
                  abuf0, bbuf0, obuf0, abuf1, bbuf1, obuf1,
                  p0all, p1all, gsem0, gsem1, ssem0, ssem1):
        wid = lax.axis_index("c") * 16 + lax.axis_index("s")
        base = wid * tok_per_w
        pltpu.sync_copy(pos_hbm.at[0, pl.ds(base, tok_per_w)], p0all)
        pltpu.sync_copy(pos_hbm.at[1, pl.ds(base, tok_per_w)], p1all)

        def issue(ck, ab, bb, gs):
            off = ck * CCHUNK
            pltpu.async_copy(y_hbm.at[p0all.at[pl.ds(off, CCHUNK)]], ab, gs)
            pltpu.async_copy(y_hbm.at[p1all.at[pl.ds(off, CCHUNK)]], bb, gs)

        def drain_gather(ab, bb, gs):
            pltpu.make_async_copy(y_hbm.at[pl.ds(0, CCHUNK), :], ab, gs).wait()
            pltpu.make_async_copy(y_hbm.at[pl.ds(0, CCHUNK), :], bb, gs).wait()

        def process(ck, ab, bb, ob, gs, ss):
            t0 = base + ck * CCHUNK
            drain_gather(ab, bb, gs)
            pltpu.sync_copy(bias_hbm.at[pl.ds(t0, CCHUNK), :], ob)
            for i in range(CCHUNK):
                def body(j, _):
                    sl = pl.ds(j * 16, 16)
                    ob[i, sl] = ob[i, sl] + (ab[i, sl] + bb[i, sl])
                    return 0
                lax.fori_loop(0, D_OUT // 16, body, 0, unroll=8)
            pltpu.async_copy(ob, out_hbm.at[pl.ds(t0, CCHUNK), :], ss)

        issue(0, abuf0, bbuf0, gsem0)

        def trip(ci2, _):
            c0 = 2 * ci2
            @pl.when(ci2 > 0)
            def _d():
                pltpu.make_async_copy(y_hbm.at[pl.ds(0, CCHUNK), :],
                                      obuf0, ssem0).wait()
                pltpu.make_async_copy(y_hbm.at[pl.ds(0, CCHUNK), :],
                                      obuf1, ssem1).wait()

            issue(c0 + 1, abuf1, bbuf1, gsem1)
            process(c0, abuf0, bbuf0, obuf0, gsem0, ssem0)

            @pl.when(ci2 < (nchunk // 2) - 1)
            def _i():
                issue(c0 + 2, abuf0, bbuf0, gsem0)

            process(c0 + 1, abuf1, bbuf1, obuf1, gsem1, ssem1)
            return 0

        lax.fori_loop(0, nchunk // 2, trip, 0)
        pltpu.make_async_copy(y_hbm.at[pl.ds(0, CCHUNK), :], obuf0, ssem0).wait()
        pltpu.make_async_copy(y_hbm.at[pl.ds(0, CCHUNK), :], obuf1, ssem1).wait()

    return combine_k


def kernel(x, d_scale, cluster_centroids, W, b):
    ds2 = d_scale.reshape(1, 1)
    c2 = cluster_centroids.reshape(E, D_IN)
    p1c, p2c, xw0, xw1, bias_term = _router(x, ds2, c2, b)

    eslot_col = jnp.concatenate([p1c, p2c], axis=0)
    pos_col, te2 = _bookkeeping(eslot_col)
    pos2 = pos_col.reshape(ES, N)
    tile_expert = te2.reshape(G)

    xs = _make_scatter()(xw0, xw1, pos2)
    Y = _grouped_gemm(xs, W, tile_expert)
    out = _make_combine()(Y, pos2, bias_term)
    return out


# speedup vs baseline: 1.0847x; 1.0847x over previous
import functools

import jax
import jax.numpy as jnp
from jax import lax
from jax.experimental import pallas as pl
from jax.experimental.pallas import tpu as pltpu
from jax.experimental.pallas import tpu_sc as plsc

N, D_IN, D_OUT, E, ES = 4096, 1024, 1024, 16, 2
TOK_TILE = 256
TK = 256
G = (ES * N) // TK + E
PADDED = G * TK
NSLOT = ES * N
CS_B = 512
CS_NB = NSLOT // CS_B

NWORKERS = 32
CHUNK = 16


def _router_body(x_ref, ds_ref, c_ref, b_ref,
                 p1_ref, p2_ref, xw0_ref, xw1_ref, bias_ref):
    xb = x_ref[...]
    s = ds_ref[0, 0]
    cols = []
    for e in range(E):
        diff = s * (xb - c_ref[e, :][None, :])
        prod = diff * diff
        acc = jnp.sum(prod[:, 0:128], axis=1)
        for k in range(1, D_IN // 128):
            acc = acc + jnp.sum(prod[:, 128 * k:128 * (k + 1)], axis=1)
        cols.append(acc * (1.0 / D_IN))
    dist = jnp.stack(cols, axis=1)

    eids = lax.broadcasted_iota(jnp.int32, dist.shape, 1)
    m1 = jnp.min(dist, axis=1, keepdims=True)
    p1 = jnp.min(jnp.where(dist == m1, eids, E), axis=1)
    dist2 = jnp.where(eids == p1[:, None], jnp.inf, dist)
    m2 = jnp.min(dist2, axis=1, keepdims=True)
    p2 = jnp.min(jnp.where(dist2 == m2, eids, E), axis=1)

    sel = dist[:, 0:ES]
    wraw = 1.0 / (1.0 + sel)
    w = wraw / jnp.sum(wraw, axis=1, keepdims=True)

    p1_ref[...] = p1[:, None]
    p2_ref[...] = p2[:, None]
    xw0_ref[...] = w[:, 0:1] * xb
    xw1_ref[...] = w[:, 1:2] * xb
    g = (w[:, 0:1] * (p1[:, None] == eids).astype(jnp.float32)
         + w[:, 1:2] * (p2[:, None] == eids).astype(jnp.float32))
    bias_ref[...] = jnp.dot(g, b_ref[...], preferred_element_type=jnp.float32)


def _router(x, ds2, c2, b):
    n_tiles = N // TOK_TILE
    return pl.pallas_call(
        _router_body,
        grid=(n_tiles,),
        in_specs=[
            pl.BlockSpec((TOK_TILE, D_IN), lambda i: (i, 0)),
            pl.BlockSpec((1, 1), lambda i: (0, 0)),
            pl.BlockSpec((E, D_IN), lambda i: (0, 0)),
            pl.BlockSpec((E, D_OUT), lambda i: (0, 0)),
        ],
        out_specs=[
            pl.BlockSpec((TOK_TILE, 1), lambda i: (i, 0)),
            pl.BlockSpec((TOK_TILE, 1), lambda i: (i, 0)),
            pl.BlockSpec((TOK_TILE, D_IN), lambda i: (i, 0)),
            pl.BlockSpec((TOK_TILE, D_IN), lambda i: (i, 0)),
            pl.BlockSpec((TOK_TILE, D_OUT), lambda i: (i, 0)),
        ],
        out_shape=[
            jax.ShapeDtypeStruct((N, 1), jnp.int32),
            jax.ShapeDtypeStruct((N, 1), jnp.int32),
            jax.ShapeDtypeStruct((N, D_IN), jnp.float32),
            jax.ShapeDtypeStruct((N, D_IN), jnp.float32),
            jax.ShapeDtypeStruct((N, D_OUT), jnp.float32),
        ],
    )(x, ds2, c2, b)


def _book_body(eslot_ref, pos_ref, te_ref):
    eslot = eslot_ref[...]
    eids = lax.broadcasted_iota(jnp.int32, (NSLOT, E), 1)
    onehot = (eslot == eids).astype(jnp.float32)

    r512 = lax.broadcasted_iota(jnp.int32, (CS_B, CS_B), 0)
    c512 = lax.broadcasted_iota(jnp.int32, (CS_B, CS_B), 1)
    trilS = (c512 < r512).astype(jnp.float32)

    bsums = []
    for bidx in range(CS_NB):
        oh_b = onehot[bidx * CS_B:(bidx + 1) * CS_B, :]
        bsums.append(jnp.sum(oh_b, axis=0))
    block_sums = jnp.stack(bsums, axis=0)

    rB = lax.broadcasted_iota(jnp.int32, (CS_NB, CS_NB), 0)
    cB = lax.broadcasted_iota(jnp.int32, (CS_NB, CS_NB), 1)
    trilB = (cB < rB).astype(jnp.float32)
    block_excl = jnp.dot(trilB, block_sums,
                         preferred_element_type=jnp.float32)

    counts = jnp.sum(block_sums, axis=0)
    ntk = jnp.floor((counts + (TK - 1)) * (1.0 / TK)) * TK
    rE = lax.broadcasted_iota(jnp.int32, (E, E), 0)
    cE = lax.broadcasted_iota(jnp.int32, (E, E), 1)
    po = jnp.sum(jnp.where(rE < cE, ntk[:, None], 0.0), axis=0)
    end_off = po + ntk

    for bidx in range(CS_NB):
        oh_b = onehot[bidx * CS_B:(bidx + 1) * CS_B, :]
        within = jnp.dot(trilS, oh_b, preferred_element_type=jnp.float32)
        texcl = within + block_excl[bidx, :][None, :] + po[None, :]
        posb = jnp.sum(oh_b * texcl, axis=1)
        pos_ref[bidx * CS_B:(bidx + 1) * CS_B, :] = posb.astype(jnp.int32)[:, None]

    tvals = (lax.broadcasted_iota(jnp.int32, (1, G), 1) * TK).astype(jnp.float32)
    te = jnp.sum((tvals >= end_off[:, None]).astype(jnp.int32), axis=0)
    te_ref[...] = jnp.minimum(te, E - 1)[None, :]


def _bookkeeping(eslot_col):
    return pl.pallas_call(
        _book_body,
        in_specs=[pl.BlockSpec((NSLOT, 1), lambda: (0, 0))],
        out_specs=[
            pl.BlockSpec((NSLOT, 1), lambda: (0, 0)),
            pl.BlockSpec((1, G), lambda: (0, 0)),
        ],
        out_shape=[
            jax.ShapeDtypeStruct((NSLOT, 1), jnp.int32),
            jax.ShapeDtypeStruct((1, G), jnp.int32),
        ],
    )(eslot_col)


SCHUNK = 32


def _make_scatter():
    mesh = plsc.VectorSubcoreMesh(core_axis_name="c", subcore_axis_name="s")
    rows_per_w = N // (NWORKERS // ES)
    nchunk = rows_per_w // SCHUNK

    @functools.partial(
        pl.kernel, mesh=mesh,
        out_type=jax.ShapeDtypeStruct((PADDED, D_IN), jnp.float32),
        scratch_types=[
            pltpu.VMEM((SCHUNK, D_IN), jnp.float32),
            pltpu.VMEM((SCHUNK, D_IN), jnp.float32),
            pltpu.VMEM((SCHUNK,), jnp.int32),
            pltpu.VMEM((SCHUNK,), jnp.int32),
            pltpu.SemaphoreType.DMA,
            pltpu.SemaphoreType.DMA,
        ],
    )
    def scatter_k(xw0_hbm, xw1_hbm, pos_hbm, xs_hbm,
                  xbuf0, xbuf1, pbuf0, pbuf1, gsem, ssem):
        wid = lax.axis_index("c") * 16 + lax.axis_index("s")
        sid = wid // 16
        base = (wid % 16) * rows_per_w
        xbufs = (xbuf0, xbuf1)
        pbufs = (pbuf0, pbuf1)

        scat = [None, None]
        for ci in range(nchunk):
            b = ci % 2
            t0 = base + ci * SCHUNK
            if scat[b] is not None:
                scat[b].wait()

            @pl.when(sid == 0)
            def _a(t0=t0, b=b):
                pltpu.sync_copy(xw0_hbm.at[pl.ds(t0, SCHUNK), :], xbufs[b])
                pltpu.sync_copy(pos_hbm.at[0, pl.ds(t0, SCHUNK)], pbufs[b])

            @pl.when(sid == 1)
            def _b(t0=t0, b=b):
                pltpu.sync_copy(xw1_hbm.at[pl.ds(t0, SCHUNK), :], xbufs[b])
                pltpu.sync_copy(pos_hbm.at[1, pl.ds(t0, SCHUNK)], pbufs[b])

            scat[b] = pltpu.async_copy(xbufs[b], xs_hbm.at[pbufs[b]], ssem)
        scat[0].wait()
        scat[1].wait()

    return scatter_k


def _gemm_body(te_ref, xs_ref, w_ref, out_ref):
    out_ref[...] = lax.dot_general(xs_ref[...], w_ref[0],
                                   (((1,), (1,)), ((), ())))


def _grouped_gemm(xs, W, tile_expert):
    grid_spec = pltpu.PrefetchScalarGridSpec(
        num_scalar_prefetch=1,
        grid=(G,),
        in_specs=[
            pl.BlockSpec((TK, D_IN), lambda i, te: (i, 0)),
            pl.BlockSpec((1, D_OUT, D_IN), lambda i, te: (te[i], 0, 0)),
        ],
        out_specs=pl.BlockSpec((TK, D_OUT), lambda i, te: (i, 0)),
    )
    return pl.pallas_call(
        _gemm_body,
        grid_spec=grid_spec,
        out_shape=jax.ShapeDtypeStruct((PADDED, D_OUT), jnp.float32),
    )(tile_expert, xs, W)


CCHUNK = 16


def _make_combine():
    mesh = plsc.VectorSubcoreMesh(core_axis_name="c", subcore_axis_name="s")
    tok_per_w = N // NWORKERS
    nchunk = tok_per_w // CCHUNK

    @functools.partial(
        pl.kernel, mesh=mesh,
        out_type=jax.ShapeDtypeStruct((N, D_OUT), jnp.float32),
        scratch_types=[
            pltpu.VMEM((CCHUNK, D_OUT), jnp.float32),
            pltpu.VMEM((CCHUNK, D_OUT), jnp.float32),
            pltpu.VMEM((CCHUNK, D_OUT), jnp.float32),
            pltpu.VMEM((CCHUNK, D_OUT), jnp.float32),
            pltpu.VMEM((CCHUNK, D_OUT), jnp.float32),
            pltpu.VMEM((CCHUNK, D_OUT), jnp.float32),
            pltpu.VMEM((tok_per_w,), jnp.int32),
            pltpu.VMEM((tok_per_w,), jnp.int32),
            pltpu.SemaphoreType.DMA,
            pltpu.SemaphoreType.DMA,
            pltpu.SemaphoreType.DMA,
            pltpu.SemaphoreType.DMA,
        ],
    )
    def combine_k(y_hbm, pos_hbm, bias_hbm, out_hbm,
                  abuf0, bbuf0, obuf0, abuf1, bbuf1, obuf1,
                  p0all, p1all, gsem0, gsem1, ssem0, ssem1):
        wid = lax.axis_index("c") * 16 + lax.axis_index("s")
        base = wid * tok_per_w
        pltpu.sync_copy(pos_hbm.at[0, pl.ds(base, tok_per_w)], p0all)
        pltpu.sync_copy(pos_hbm.at[1, pl.ds(base, tok_per_w)], p1all)

        def issue(ck, ab, bb, gs):
            off = ck * CCHUNK
            pltpu.async_copy(y_hbm.at[p0all.at[pl.ds(off, CCHUNK)]], ab, gs)
            pltpu.async_copy(y_hbm.at[p1all.at[pl.ds(off, CCHUNK)]], bb, gs)

        def drain_gather(ab, bb, gs):
            pltpu.make_async_copy(y_hbm.at[pl.ds(0, CCHUNK), :], ab, gs).wait()
            pltpu.make_async_copy(y_hbm.at[pl.ds(0, CCHUNK), :], bb, gs).wait()

        def process(ck, ab, bb, ob, gs, ss):
            t0 = base + ck * CCHUNK
            drain_gather(ab, bb, gs)
            pltpu.sync_copy(bias_hbm.at[pl.ds(t0, CCHUNK), :], ob)
            for i in range(CCHUNK):
                def body(j, _):
                    sl = pl.ds(j * 16, 16)
                    ob[i, sl] = ob[i, sl] + (ab[i, sl] + bb[i, sl])
                    return 0
                lax.fori_loop(0, D_OUT // 16, body, 0, unroll=8)
            pltpu.async_copy(ob, out_hbm.at[pl.ds(t0, CCHUNK), :], ss)

        issue(0, abuf0, bbuf0, gsem0)

        def trip(ci2, _):
            c0 = 2 * ci2
            @pl.when(ci2 > 0)
            def _d():
                pltpu.make_async_copy(y_hbm.at[pl.ds(0, CCHUNK), :],
                                      obuf0, ssem0).wait()
                pltpu.make_async_copy(y_hbm.at[pl.ds(0, CCHUNK), :],
                                      obuf1, ssem1).wait()

            issue(c0 + 1, abuf1, bbuf1, gsem1)
            process(c0, abuf0, bbuf0, obuf0, gsem0, ssem0)

            @pl.when(ci2 < (nchunk // 2) - 1)
            def _i():
                issue(c0 + 2, abuf0, bbuf0, gsem0)

            process(c0 + 1, abuf1, bbuf1, obuf1, gsem1, ssem1)
            return 0

        lax.fori_loop(0, nchunk // 2, trip, 0)
        pltpu.make_async_copy(y_hbm.at[pl.ds(0, CCHUNK), :], obuf0, ssem0).wait()
        pltpu.make_async_copy(y_hbm.at[pl.ds(0, CCHUNK), :], obuf1, ssem1).wait()

    return combine_k


def kernel(x, d_scale, cluster_centroids, W, b):
    ds2 = d_scale.reshape(1, 1)
    c2 = cluster_centroids.reshape(E, D_IN)
    p1c, p2c, xw0, xw1, bias_term = _router(x, ds2, c2, b)

    eslot_col = jnp.concatenate([p1c, p2c], axis=0)
    pos_col, te2 = _bookkeeping(eslot_col)
    pos2 = pos_col.reshape(ES, N)
    tile_expert = te2.reshape(G)

    xs = _make_scatter()(xw0, xw1, pos2)
    Y = _grouped_gemm(xs, W, tile_expert)
    out = _make_combine()(Y, pos2, bias_term)
    return out


# speedup vs baseline: 1.1790x; 1.0870x over previous
import functools

import jax
import jax.numpy as jnp
from jax import lax
from jax.experimental import pallas as pl
from jax.experimental.pallas import tpu as pltpu
from jax.experimental.pallas import tpu_sc as plsc

N, D_IN, D_OUT, E, ES = 4096, 1024, 1024, 16, 2
TOK_TILE = 256
TK = 256
G = (ES * N) // TK + E
PADDED = G * TK
NSLOT = ES * N
CS_B = 512
CS_NB = NSLOT // CS_B

NWORKERS = 32
CHUNK = 16


def _router_body(x_ref, ds_ref, c_ref, b_ref,
                 xw0_ref, xw1_ref, bias_ref, pos_ref, te_ref, eslot_scr):
    i = pl.program_id(0)
    xb = x_ref[...]
    s = ds_ref[0, 0]
    cols = []
    for e in range(E):
        diff = s * (xb - c_ref[e, :][None, :])
        prod = diff * diff
        acc = jnp.sum(prod[:, 0:128], axis=1)
        for k in range(1, D_IN // 128):
            acc = acc + jnp.sum(prod[:, 128 * k:128 * (k + 1)], axis=1)
        cols.append(acc * (1.0 / D_IN))
    dist = jnp.stack(cols, axis=1)

    eids = lax.broadcasted_iota(jnp.int32, dist.shape, 1)
    m1 = jnp.min(dist, axis=1, keepdims=True)
    p1 = jnp.min(jnp.where(dist == m1, eids, E), axis=1)
    dist2 = jnp.where(eids == p1[:, None], jnp.inf, dist)
    m2 = jnp.min(dist2, axis=1, keepdims=True)
    p2 = jnp.min(jnp.where(dist2 == m2, eids, E), axis=1)

    sel = dist[:, 0:ES]
    wraw = 1.0 / (1.0 + sel)
    w = wraw / jnp.sum(wraw, axis=1, keepdims=True)

    eslot_scr[pl.ds(i * TOK_TILE, TOK_TILE), :] = p1[:, None]
    eslot_scr[pl.ds(N + i * TOK_TILE, TOK_TILE), :] = p2[:, None]
    xw0_ref[...] = w[:, 0:1] * xb
    xw1_ref[...] = w[:, 1:2] * xb
    g = (w[:, 0:1] * (p1[:, None] == eids).astype(jnp.float32)
         + w[:, 1:2] * (p2[:, None] == eids).astype(jnp.float32))
    bias_ref[...] = jnp.dot(g, b_ref[...], preferred_element_type=jnp.float32)

    @pl.when(i == (N // TOK_TILE) - 1)
    def _book():
        _book_compute(eslot_scr, pos_ref, te_ref)


def _router(x, ds2, c2, b):
    n_tiles = N // TOK_TILE
    return pl.pallas_call(
        _router_body,
        grid=(n_tiles,),
        in_specs=[
            pl.BlockSpec((TOK_TILE, D_IN), lambda i: (i, 0)),
            pl.BlockSpec((1, 1), lambda i: (0, 0)),
            pl.BlockSpec((E, D_IN), lambda i: (0, 0)),
            pl.BlockSpec((E, D_OUT), lambda i: (0, 0)),
        ],
        out_specs=[
            pl.BlockSpec((TOK_TILE, D_IN), lambda i: (i, 0)),
            pl.BlockSpec((TOK_TILE, D_IN), lambda i: (i, 0)),
            pl.BlockSpec((TOK_TILE, D_OUT), lambda i: (i, 0)),
            pl.BlockSpec((NSLOT, 1), lambda i: (0, 0)),
            pl.BlockSpec((1, G), lambda i: (0, 0)),
        ],
        out_shape=[
            jax.ShapeDtypeStruct((N, D_IN), jnp.float32),
            jax.ShapeDtypeStruct((N, D_IN), jnp.float32),
            jax.ShapeDtypeStruct((N, D_OUT), jnp.float32),
            jax.ShapeDtypeStruct((NSLOT, 1), jnp.int32),
            jax.ShapeDtypeStruct((1, G), jnp.int32),
        ],
        scratch_shapes=[pltpu.VMEM((NSLOT, 1), jnp.int32)],
    )(x, ds2, c2, b)


def _book_compute(eslot_ref, pos_ref, te_ref):
    eslot = eslot_ref[...]
    eids = lax.broadcasted_iota(jnp.int32, (NSLOT, E), 1)
    onehot = (eslot == eids).astype(jnp.float32)

    r512 = lax.broadcasted_iota(jnp.int32, (CS_B, CS_B), 0)
    c512 = lax.broadcasted_iota(jnp.int32, (CS_B, CS_B), 1)
    trilS = (c512 < r512).astype(jnp.float32)

    bsums = []
    for bidx in range(CS_NB):
        oh_b = onehot[bidx * CS_B:(bidx + 1) * CS_B, :]
        bsums.append(jnp.sum(oh_b, axis=0))
    block_sums = jnp.stack(bsums, axis=0)

    rB = lax.broadcasted_iota(jnp.int32, (CS_NB, CS_NB), 0)
    cB = lax.broadcasted_iota(jnp.int32, (CS_NB, CS_NB), 1)
    trilB = (cB < rB).astype(jnp.float32)
    block_excl = jnp.dot(trilB, block_sums,
                         preferred_element_type=jnp.float32)

    counts = jnp.sum(block_sums, axis=0)
    ntk = jnp.floor((counts + (TK - 1)) * (1.0 / TK)) * TK
    rE = lax.broadcasted_iota(jnp.int32, (E, E), 0)
    cE = lax.broadcasted_iota(jnp.int32, (E, E), 1)
    po = jnp.sum(jnp.where(rE < cE, ntk[:, None], 0.0), axis=0)
    end_off = po + ntk

    for bidx in range(CS_NB):
        oh_b = onehot[bidx * CS_B:(bidx + 1) * CS_B, :]
        within = jnp.dot(trilS, oh_b, preferred_element_type=jnp.float32)
        texcl = within + block_excl[bidx, :][None, :] + po[None, :]
        posb = jnp.sum(oh_b * texcl, axis=1)
        pos_ref[bidx * CS_B:(bidx + 1) * CS_B, :] = posb.astype(jnp.int32)[:, None]

    tvals = (lax.broadcasted_iota(jnp.int32, (1, G), 1) * TK).astype(jnp.float32)
    te = jnp.sum((tvals >= end_off[:, None]).astype(jnp.int32), axis=0)
    te_ref[...] = jnp.minimum(te, E - 1)[None, :]


SCHUNK = 32


def _make_scatter():
    mesh = plsc.VectorSubcoreMesh(core_axis_name="c", subcore_axis_name="s")
    rows_per_w = N // (NWORKERS // ES)
    nchunk = rows_per_w // SCHUNK

    @functools.partial(
        pl.kernel, mesh=mesh,
        out_type=jax.ShapeDtypeStruct((PADDED, D_IN), jnp.float32),
        scratch_types=[
            pltpu.VMEM((SCHUNK, D_IN), jnp.float32),
            pltpu.VMEM((SCHUNK, D_IN), jnp.float32),
            pltpu.VMEM((SCHUNK,), jnp.int32),
            pltpu.VMEM((SCHUNK,), jnp.int32),
            pltpu.SemaphoreType.DMA,
            pltpu.SemaphoreType.DMA,
        ],
    )
    def scatter_k(xw0_hbm, xw1_hbm, pos_hbm, xs_hbm,
                  xbuf0, xbuf1, pbuf0, pbuf1, gsem, ssem):
        wid = lax.axis_index("c") * 16 + lax.axis_index("s")
        sid = wid // 16
        base = (wid % 16) * rows_per_w
        xbufs = (xbuf0, xbuf1)
        pbufs = (pbuf0, pbuf1)

        scat = [None, None]
        for ci in range(nchunk):
            b = ci % 2
            t0 = base + ci * SCHUNK
            if scat[b] is not None:
                scat[b].wait()

            @pl.when(sid == 0)
            def _a(t0=t0, b=b):
                pltpu.sync_copy(xw0_hbm.at[pl.ds(t0, SCHUNK), :], xbufs[b])
                pltpu.sync_copy(pos_hbm.at[0, pl.ds(t0, SCHUNK)], pbufs[b])

            @pl.when(sid == 1)
            def _b(t0=t0, b=b):
                pltpu.sync_copy(xw1_hbm.at[pl.ds(t0, SCHUNK), :], xbufs[b])
                pltpu.sync_copy(pos_hbm.at[1, pl.ds(t0, SCHUNK)], pbufs[b])

            scat[b] = pltpu.async_copy(xbufs[b], xs_hbm.at[pbufs[b]], ssem)
        scat[0].wait()
        scat[1].wait()

    return scatter_k


def _gemm_body(te_ref, xs_ref, w_ref, out_ref):
    out_ref[...] = lax.dot_general(xs_ref[...], w_ref[0],
                                   (((1,), (1,)), ((), ())))


def _grouped_gemm(xs, W, tile_expert):
    grid_spec = pltpu.PrefetchScalarGridSpec(
        num_scalar_prefetch=1,
        grid=(G,),
        in_specs=[
            pl.BlockSpec((TK, D_IN), lambda i, te: (i, 0)),
            pl.BlockSpec((1, D_OUT, D_IN), lambda i, te: (te[i], 0, 0)),
        ],
        out_specs=pl.BlockSpec((TK, D_OUT), lambda i, te: (i, 0)),
    )
    return pl.pallas_call(
        _gemm_body,
        grid_spec=grid_spec,
        out_shape=jax.ShapeDtypeStruct((PADDED, D_OUT), jnp.float32),
    )(tile_expert, xs, W)


CCHUNK = 32


def _make_combine():
    mesh = plsc.VectorSubcoreMesh(core_axis_name="c", subcore_axis_name="s")
    tok_per_w = N // NWORKERS

    @functools.partial(
        pl.kernel, mesh=mesh,
        out_type=jax.ShapeDtypeStruct((N, D_OUT), jnp.float32),
        scratch_types=[
            pltpu.VMEM((CCHUNK, D_OUT), jnp.float32),
            pltpu.VMEM((CCHUNK, D_OUT), jnp.float32),
            pltpu.VMEM((CCHUNK, D_OUT), jnp.float32),
            pltpu.VMEM((CCHUNK,), jnp.int32),
            pltpu.VMEM((CCHUNK,), jnp.int32),
            pltpu.SemaphoreType.DMA,
        ],
    )
    def combine_k(y_hbm, pos_hbm, bias_hbm, out_hbm,
                  abuf, bbuf, obuf, p0buf, p1buf, sem):
        wid = lax.axis_index("c") * 16 + lax.axis_index("s")
        base = wid * tok_per_w

        def chunk_body(ci, _):
            t0 = base + ci * CCHUNK
            pltpu.sync_copy(pos_hbm.at[0, pl.ds(t0, CCHUNK)], p0buf)
            pltpu.sync_copy(pos_hbm.at[1, pl.ds(t0, CCHUNK)], p1buf)
            cpa = pltpu.async_copy(y_hbm.at[p0buf], abuf, sem)
            cpb = pltpu.async_copy(y_hbm.at[p1buf], bbuf, sem)
            pltpu.sync_copy(bias_hbm.at[pl.ds(t0, CCHUNK), :], obuf)
            cpa.wait()
            cpb.wait()
            for i in range(CCHUNK):
                def body(j, _):
                    sl = pl.ds(j * 16, 16)
                    obuf[i, sl] = obuf[i, sl] + (abuf[i, sl] + bbuf[i, sl])
                    return 0
                lax.fori_loop(0, D_OUT // 16, body, 0, unroll=8)
            pltpu.sync_copy(obuf, out_hbm.at[pl.ds(t0, CCHUNK), :])
            return 0

        lax.fori_loop(0, tok_per_w // CCHUNK, chunk_body, 0)

    return combine_k


def kernel(x, d_scale, cluster_centroids, W, b):
    ds2 = d_scale.reshape(1, 1)
    c2 = cluster_centroids.reshape(E, D_IN)
    xw0, xw1, bias_term, pos_col, te2 = _router(x, ds2, c2, b)
    pos2 = pos_col.reshape(ES, N)
    tile_expert = te2.reshape(G)

    xs = _make_scatter()(xw0, xw1, pos2)
    Y = _grouped_gemm(xs, W, tile_expert)
    out = _make_combine()(Y, pos2, bias_term)
    return out
